# baseline (speedup 1.0000x reference)
import functools

import jax
import jax.numpy as jnp
from jax import lax
from jax.experimental import pallas as pl
from jax.experimental.pallas import tpu as pltpu
from jax.experimental.pallas import tpu_sc as plsc

_CHUNK = 128


def _make_gather(B, D, num_cores, num_subcores):
    nw = num_cores * num_subcores
    b_per_w = B // nw
    n_chunks = b_per_w // _CHUNK
    mesh = plsc.VectorSubcoreMesh(core_axis_name="c", subcore_axis_name="s")

    @functools.partial(
        pl.kernel,
        mesh=mesh,
        compiler_params=pltpu.CompilerParams(use_tc_tiling_on_sc=False),
        out_type=jax.ShapeDtypeStruct((B, D), jnp.float32),
        scratch_types=[
            pltpu.VMEM((n_chunks, _CHUNK), jnp.int32),
            pltpu.VMEM((n_chunks, _CHUNK, D), jnp.float32),
            pltpu.SemaphoreType.DMA,
        ],
    )
    def gather(idx_hbm, table_hbm, out_hbm, idx_v, rows_v, sem):
        wid = lax.axis_index("s") * num_cores + lax.axis_index("c")
        base = wid * b_per_w
        idx_copies = [
            pltpu.async_copy(
                idx_hbm.at[pl.ds(base + j * _CHUNK, _CHUNK)], idx_v.at[j], sem
            )
            for j in range(n_chunks)
        ]
        for c in idx_copies:
            c.wait()
        row_copies = [
            pltpu.async_copy(table_hbm.at[idx_v.at[j]], rows_v.at[j], sem)
            for j in range(n_chunks)
        ]
        for c in row_copies:
            c.wait()
        out_copies = [
            pltpu.async_copy(
                rows_v.at[j], out_hbm.at[pl.ds(base + j * _CHUNK, _CHUNK)], sem
            )
            for j in range(n_chunks)
        ]
        for c in out_copies:
            c.wait()

    return gather


def _proj_body(x_ref, wt_ref, b_ref, out_ref):
    out_ref[...] = (
        jnp.dot(x_ref[...], wt_ref[...], preferred_element_type=jnp.float32)
        + b_ref[...]
    )


def _proj(embs, Wt, b2d):
    B, D = embs.shape
    O = Wt.shape[1]
    blk = 2048
    return pl.pallas_call(
        _proj_body,
        grid=(B // blk,),
        in_specs=[
            pl.BlockSpec((blk, D), lambda i: (i, 0)),
            pl.BlockSpec((D, O), lambda i: (0, 0)),
            pl.BlockSpec((1, O), lambda i: (0, 0)),
        ],
        out_specs=pl.BlockSpec((blk, O), lambda i: (i, 0)),
        out_shape=jax.ShapeDtypeStruct((B, O), jnp.float32),
    )(embs, Wt, b2d)


def kernel(indices, table, W, b):
    info = plsc.get_sparse_core_info()
    embs = _make_gather(indices.shape[0], table.shape[1],
                        info.num_cores, info.num_subcores)(
        indices.astype(jnp.int32), table
    )
    return _proj(embs, W.T, b.reshape(1, -1))
